# baseline (speedup 1.0000x reference)
import functools

import jax
import jax.numpy as jnp
from jax import lax
from jax.experimental import pallas as pl
from jax.experimental.pallas import tpu as pltpu
from jax.experimental.pallas import tpu_sc as plsc

B = 4
N = 2048
KN = 30
KP = 32
DN = 256
DE = 128

BLKA = 512
BLKB = 512
BLKC = 256
BLKD = 128

def _softplus(x):
    return jnp.maximum(x, 0.0) + jnp.log1p(jnp.exp(-jnp.abs(x)))


def _knn_body(bb, cen_ref, cent_ref, c_ref, vals_ref, idxp_ref, idxg_ref):
    b = bb
    i = pl.program_id(0)
    r0 = i * BLKA

    cols_c = c_ref[0]
    col_iota = lax.broadcasted_iota(jnp.int32, (1, N), 1)
    rows = r0 + lax.broadcasted_iota(jnp.int32, (BLKA, 1), 0)

    acc = None
    for d in range(3):
        diff = cen_ref[0, :, d:d + 1] - cent_ref[0, d:d + 1, :]
        acc = diff * diff if acc is None else acc + diff * diff
    dist = jnp.sqrt(acc + 1e-8)
    dist = dist + jnp.where(col_iota == rows, 1e6, 0.0)
    dist = dist + jnp.where(cols_c > 0, 0.0, 1e6)

    ip = col_iota * 4 + cols_c
    ip_b = jnp.broadcast_to(ip, (BLKA, N))
    lane = lax.broadcasted_iota(jnp.int32, (1, KP), 1)

    dc = dist
    vals = jnp.zeros((BLKA, KP), jnp.float32)
    idxp = jnp.zeros((BLKA, KP), jnp.int32)
    for k in range(KN):
        m = jnp.min(dc, axis=1, keepdims=True)
        pk = jnp.min(jnp.where(dc == m, ip_b, jnp.int32(1 << 30)),
                     axis=1, keepdims=True)
        dc = jnp.where(ip_b == pk, 1e9, dc)
        vals = jnp.where(lane == k, m, vals)
        idxp = jnp.where(lane == k, pk, idxp)
    vals_ref[0] = vals
    idxp_ref[0] = idxp
    idxg_ref[0] = (lax.shift_right_logical(idxp, 2) + b * N)[:, :KN]


def _knn(bb, cen, cent, c2):
    return pl.pallas_call(
        functools.partial(_knn_body, bb),
        grid=(N // BLKA,),
        in_specs=[
            pl.BlockSpec((1, BLKA, 3), lambda i, bb=bb: (bb, i, 0)),
            pl.BlockSpec((1, 3, N), lambda i, bb=bb: (bb, 0, 0)),
            pl.BlockSpec((1, 1, N), lambda i, bb=bb: (bb, 0, 0)),
        ],
        out_specs=[
            pl.BlockSpec((1, BLKA, KP), lambda i: (0, i, 0)),
            pl.BlockSpec((1, BLKA, KP), lambda i: (0, i, 0)),
            pl.BlockSpec((1, BLKA, KN), lambda i: (0, i, 0)),
        ],
        out_shape=[
            jax.ShapeDtypeStruct((1, N, KP), jnp.float32),
            jax.ShapeDtypeStruct((1, N, KP), jnp.int32),
            jax.ShapeDtypeStruct((1, N, KN), jnp.int32),
        ],
    )(cen, cent, c2)


def _node_body(x_ref, c_ref, wn_ref, bn_ref, w1_ref, w2_ref,
               nh_ref, p1_ref, p2_ref, m_ref):
    x = x_ref[:, :]
    mask = (c_ref[:, :] > 0).astype(jnp.float32)
    h = jnp.dot(x, wn_ref[:, :], preferred_element_type=jnp.float32)
    h = _softplus(h + bn_ref[0][None, :]) * mask
    nh_ref[:, :] = h
    p1_ref[:, :] = jnp.dot(h, w1_ref[:, :], preferred_element_type=jnp.float32)
    p2 = jnp.dot(h, w2_ref[:, :], preferred_element_type=jnp.float32)
    def _rne_hi16(x):
        bits = lax.bitcast_convert_type(x, jnp.int32)
        lsb = jnp.bitwise_and(lax.shift_right_logical(bits, 16), 1)
        return jnp.bitwise_and(bits + 32767 + lsb, jnp.int32(-65536))

    lo = lax.shift_right_logical(_rne_hi16(p2[:, :DN // 2]), 16)
    hi = _rne_hi16(p2[:, DN // 2:])
    p2_ref[:, :] = jnp.bitwise_or(hi, lo)
    m_ref[:, :] = mask


def _node(xf, cf, w_node, bn2, w1, w2):
    nb = (B * N) // BLKB
    return pl.pallas_call(
        _node_body,
        grid=(nb,),
        in_specs=[
            pl.BlockSpec((BLKB, 12), lambda i: (i, 0)),
            pl.BlockSpec((BLKB, 1), lambda i: (i, 0)),
            pl.BlockSpec((12, DN), lambda i: (0, 0)),
            pl.BlockSpec((1, DN), lambda i: (0, 0)),
            pl.BlockSpec((DN, DN), lambda i: (0, 0)),
            pl.BlockSpec((DN, DN), lambda i: (0, 0)),
        ],
        out_specs=[
            pl.BlockSpec((BLKB, DN), lambda i: (i, 0)),
            pl.BlockSpec((BLKB, DN), lambda i: (i, 0)),
            pl.BlockSpec((BLKB, DN // 2), lambda i: (i, 0)),
            pl.BlockSpec((BLKB, 1), lambda i: (i, 0)),
        ],
        out_shape=[
            jax.ShapeDtypeStruct((B * N, DN), jnp.float32),
            jax.ShapeDtypeStruct((B * N, DN), jnp.float32),
            jax.ShapeDtypeStruct((B * N, DN // 2), jnp.int32),
            jax.ShapeDtypeStruct((B * N, 1), jnp.float32),
        ],
    )(xf, cf, w_node, bn2, w1, w2)


def _edge_msg_body(vals_ref, idxp_ref, c_ref, mi_ref, we_ref, be_ref,
                   hj_ref, p1_ref, nh_ref, w3_ref, bm_ref,
                   eh_ref, mij_ref, eidx_ref, out_ref):
    sigma = 20.0 / 16.0
    centers = lax.broadcasted_iota(jnp.int32, (1, 16), 1).astype(
        jnp.float32) * (20.0 / 15.0)
    c_i = c_ref[0]
    m_i = mi_ref[0]
    vals = vals_ref[0]
    idxp = idxp_ref[0]
    cj = jnp.bitwise_and(idxp, 3)
    mij = m_i * (cj > 0).astype(jnp.float32)
    same = (c_i == cj).astype(jnp.float32)
    be = be_ref[0][None, :]
    w_same = we_ref[16:17, :]
    p1 = p1_ref[0]
    bm = bm_ref[0][None, :]
    acc = jnp.zeros((BLKC, DN), jnp.float32)
    G = 6
    for g in range(KN // G):
        ks = range(g * G, (g + 1) * G)
        dg = jnp.concatenate([vals[:, k:k + 1] for k in ks], axis=0)
        sg = jnp.concatenate([same[:, k:k + 1] for k in ks], axis=0)
        mg = jnp.concatenate([mij[:, k:k + 1] for k in ks], axis=0)
        q = dg - centers
        rbf = jnp.exp(-(q * q) / (2.0 * sigma * sigma))
        lin = jnp.dot(rbf, we_ref[0:16, :],
                      preferred_element_type=jnp.float32)
        lin = lin + sg * w_same + be
        ehg = _softplus(lin) * mg
        for j, k in enumerate(ks):
            eh_ref[0, :, k, :] = ehg[j * BLKC:(j + 1) * BLKC, :]
        hjg = jnp.concatenate([hj_ref[0, :, k, :] for k in ks], axis=0)
        hj_lo = lax.bitcast_convert_type(
            lax.shift_left(hjg, 16), jnp.float32)
        hj_hi = lax.bitcast_convert_type(
            jnp.bitwise_and(hjg, jnp.int32(-65536)), jnp.float32)
        z = jnp.dot(ehg.astype(jnp.bfloat16), w3_ref[:, :],
                    preferred_element_type=jnp.float32)
        z = z + jnp.concatenate([hj_lo, hj_hi], axis=1) + bm
        sp = _softplus(z + jnp.concatenate([p1] * G, axis=0)) * mg
        for j in range(G):
            acc = acc + sp[j * BLKC:(j + 1) * BLKC, :]
    mij_ref[0] = mij[:, :KN]
    eidx_ref[0] = lax.shift_right_logical(idxp, 2)[:, :KN]
    cnt = jnp.sum(mij[:, :KN], axis=1, keepdims=True)
    agg = acc / (cnt + 1e-6)
    out_ref[0] = (nh_ref[0] + agg) * m_i


def _edge_msg(bb, vals, idxp, cn, mi, w_edge, be2, hj, p1, nh, w3, bm2):
    return pl.pallas_call(
        _edge_msg_body,
        grid=(N // BLKC,),
        in_specs=[
            pl.BlockSpec((1, BLKC, KP), lambda i: (0, i, 0)),
            pl.BlockSpec((1, BLKC, KP), lambda i: (0, i, 0)),
            pl.BlockSpec((1, BLKC, 1), lambda i, bb=bb: (bb, i, 0)),
            pl.BlockSpec((1, BLKC, 1), lambda i, bb=bb: (bb, i, 0)),
            pl.BlockSpec((17, DE), lambda i: (0, 0)),
            pl.BlockSpec((1, DE), lambda i: (0, 0)),
            pl.BlockSpec((1, BLKC, KN, DN // 2), lambda i: (0, i, 0, 0)),
            pl.BlockSpec((1, BLKC, DN), lambda i, bb=bb: (bb, i, 0)),
            pl.BlockSpec((1, BLKC, DN), lambda i, bb=bb: (bb, i, 0)),
            pl.BlockSpec((DE, DN), lambda i: (0, 0)),
            pl.BlockSpec((1, DN), lambda i: (0, 0)),
        ],
        out_specs=[
            pl.BlockSpec((1, BLKC, KN, DE), lambda i: (0, i, 0, 0)),
            pl.BlockSpec((1, BLKC, KN), lambda i: (0, i, 0)),
            pl.BlockSpec((1, BLKC, KN), lambda i: (0, i, 0)),
            pl.BlockSpec((1, BLKC, DN), lambda i: (0, i, 0)),
        ],
        out_shape=[
            jax.ShapeDtypeStruct((1, N, KN, DE), jnp.float32),
            jax.ShapeDtypeStruct((1, N, KN), jnp.float32),
            jax.ShapeDtypeStruct((1, N, KN), jnp.int32),
            jax.ShapeDtypeStruct((1, N, DN), jnp.float32),
        ],
    )(vals, idxp, cn, mi, w_edge, be2, hj, p1, nh, w3, bm2)


_NC, _NS = 2, 16
_NW = _NC * _NS
_EP = N * KN
_B_PER_W = _EP // _NW
_CH = 128
_NCH = _B_PER_W // _CH


_NBUF = 6


def _sc_gather_body(tbl_ref, idx_ref, out_ref, idx_v, *bufs_sems):
    bufs = bufs_sems[:_NBUF]
    gsem = bufs_sems[_NBUF:2 * _NBUF]
    osem = bufs_sems[2 * _NBUF:]
    wid = lax.axis_index("s") * _NC + lax.axis_index("c")
    base = wid * _B_PER_W
    pltpu.sync_copy(idx_ref.at[pl.ds(base, _B_PER_W)], idx_v)

    def fire_gather(ci):
        j = ci % _NBUF
        return pltpu.async_copy(
            tbl_ref.at[idx_v.at[pl.ds(ci * _CH, _CH)]], bufs[j], gsem[j])

    gpend, opend = {}, {}
    for ci in range(3):
        gpend[ci] = fire_gather(ci)
    for ci in range(_NCH):
        j = ci % _NBUF
        nxt = ci + 3
        if nxt < _NCH:
            if nxt - _NBUF >= 0:
                opend[nxt - _NBUF].wait()
            gpend[nxt] = fire_gather(nxt)
        gpend[ci].wait()
        opend[ci] = pltpu.async_copy(
            bufs[j], out_ref.at[pl.ds(base + ci * _CH, _CH)], osem[j])
    for ci in range(_NCH - _NBUF, _NCH):
        opend[ci].wait()


@functools.cache
def _sc_gather():
    return pl.kernel(
        _sc_gather_body,
        out_type=jax.ShapeDtypeStruct((_EP, DN // 2), jnp.int32),
        mesh=plsc.VectorSubcoreMesh(core_axis_name="c", subcore_axis_name="s",
                                    num_cores=_NC, num_subcores=_NS),
        scratch_types=(
            [pltpu.VMEM((_B_PER_W,), jnp.int32)]
            + [pltpu.VMEM((_CH, DN // 2), jnp.int32) for _ in range(_NBUF)]
            + [pltpu.SemaphoreType.DMA for _ in range(2 * _NBUF)]
        ),
    )


def _gather_rows(table, idxg):
    return _sc_gather()(table, idxg)


def kernel(X, C, W_node, b_node, W_edge, b_edge, W_msg, b_msg):
    X = X.astype(jnp.float32)
    C32 = C.astype(jnp.int32)

    centroid = X.mean(axis=2)
    cent = centroid.transpose(0, 2, 1)
    c2 = C32.reshape(B, 1, N)

    xf = X.reshape(B * N, 12)
    cf = C32.reshape(B * N, 1)
    w1 = W_msg[:DN, :]
    w2 = W_msg[DN:2 * DN, :]
    w3 = W_msg[2 * DN:, :].astype(jnp.bfloat16)
    nh, p1, p2, mi = _node(xf, cf, W_node, b_node.reshape(1, DN), w1, w2)

    cn = C32.reshape(B, N, 1)
    mi3 = mi.reshape(B, N, 1)
    p13 = p1.reshape(B, N, DN)
    nh3 = nh.reshape(B, N, DN)
    be2 = b_edge.reshape(1, DE)
    bm2 = b_msg.reshape(1, DN)

    knn_out = [_knn(bb, centroid, cent, c2) for bb in range(B)]
    hjs = [_gather_rows(p2, knn_out[bb][2].reshape(_EP)) for bb in range(B)]
    ems = [
        _edge_msg(bb, knn_out[bb][0], knn_out[bb][1], cn, mi3, W_edge, be2,
                  hjs[bb].reshape(1, N, KN, DN // 2), p13, nh3, w3, bm2)
        for bb in range(B)
    ]
    eh = jnp.concatenate([e[0] for e in ems], axis=0)
    mij = jnp.concatenate([e[1] for e in ems], axis=0)
    eidx = jnp.concatenate([e[2] for e in ems], axis=0)
    node_h = jnp.concatenate([e[3] for e in ems], axis=0)

    return (node_h, eh, eidx, mi.reshape(B, N), mij)


# speedup vs baseline: 1.0637x; 1.0637x over previous
import functools

import jax
import jax.numpy as jnp
from jax import lax
from jax.experimental import pallas as pl
from jax.experimental.pallas import tpu as pltpu
from jax.experimental.pallas import tpu_sc as plsc

B = 4
N = 2048
KN = 30
KP = 32
DN = 256
DE = 128

BLKA = 512
BLKB = 512
BLKC = 256
BLKD = 128

def _softplus(x):
    return jnp.maximum(x, 0.0) + jnp.log1p(jnp.exp(-jnp.abs(x)))


def _knn_body(bb, cen_ref, cent_ref, c_ref, vals_ref, idxp_ref, idxg_ref):
    b = bb
    i = pl.program_id(0)
    r0 = i * BLKA

    cols_c = c_ref[0]
    col_iota = lax.broadcasted_iota(jnp.int32, (1, N), 1)
    rows = r0 + lax.broadcasted_iota(jnp.int32, (BLKA, 1), 0)

    acc = None
    for d in range(3):
        diff = cen_ref[0, :, d:d + 1] - cent_ref[0, d:d + 1, :]
        acc = diff * diff if acc is None else acc + diff * diff
    dist = jnp.sqrt(acc + 1e-8)
    dist = dist + jnp.where(col_iota == rows, 1e6, 0.0)
    dist = dist + jnp.where(cols_c > 0, 0.0, 1e6)

    ip = col_iota * 4 + cols_c
    ip_b = jnp.broadcast_to(ip, (BLKA, N))
    lane = lax.broadcasted_iota(jnp.int32, (1, KP), 1)

    dc = dist
    vals = jnp.zeros((BLKA, KP), jnp.float32)
    idxp = jnp.zeros((BLKA, KP), jnp.int32)
    for k in range(KN):
        m = jnp.min(dc, axis=1, keepdims=True)
        pk = jnp.min(jnp.where(dc == m, ip_b, jnp.int32(1 << 30)),
                     axis=1, keepdims=True)
        dc = jnp.where(ip_b == pk, 1e9, dc)
        vals = jnp.where(lane == k, m, vals)
        idxp = jnp.where(lane == k, pk, idxp)
    vals_ref[0] = vals
    idxp_ref[0] = idxp
    idxg_ref[0] = (lax.shift_right_logical(idxp, 2) + b * N)[:, :KN]


def _knn(bb, cen, cent, c2):
    return pl.pallas_call(
        functools.partial(_knn_body, bb),
        grid=(N // BLKA,),
        in_specs=[
            pl.BlockSpec((1, BLKA, 3), lambda i, bb=bb: (bb, i, 0)),
            pl.BlockSpec((1, 3, N), lambda i, bb=bb: (bb, 0, 0)),
            pl.BlockSpec((1, 1, N), lambda i, bb=bb: (bb, 0, 0)),
        ],
        out_specs=[
            pl.BlockSpec((1, BLKA, KP), lambda i: (0, i, 0)),
            pl.BlockSpec((1, BLKA, KP), lambda i: (0, i, 0)),
            pl.BlockSpec((1, BLKA, KN), lambda i: (0, i, 0)),
        ],
        out_shape=[
            jax.ShapeDtypeStruct((1, N, KP), jnp.float32),
            jax.ShapeDtypeStruct((1, N, KP), jnp.int32),
            jax.ShapeDtypeStruct((1, N, KN), jnp.int32),
        ],
    )(cen, cent, c2)


def _node_body(x_ref, c_ref, wn_ref, bn_ref, w1_ref, w2_ref,
               nh_ref, p1_ref, p2_ref, m_ref):
    x = x_ref[:, :]
    mask = (c_ref[:, :] > 0).astype(jnp.float32)
    h = jnp.dot(x, wn_ref[:, :], preferred_element_type=jnp.float32)
    h = _softplus(h + bn_ref[0][None, :]) * mask
    nh_ref[:, :] = h
    p1_ref[:, :] = jnp.dot(h, w1_ref[:, :], preferred_element_type=jnp.float32)
    p2 = jnp.dot(h, w2_ref[:, :], preferred_element_type=jnp.float32)
    def _rne_hi16(x):
        bits = lax.bitcast_convert_type(x, jnp.int32)
        lsb = jnp.bitwise_and(lax.shift_right_logical(bits, 16), 1)
        return jnp.bitwise_and(bits + 32767 + lsb, jnp.int32(-65536))

    lo = lax.shift_right_logical(_rne_hi16(p2[:, :DN // 2]), 16)
    hi = _rne_hi16(p2[:, DN // 2:])
    p2_ref[:, :] = jnp.bitwise_or(hi, lo)
    m_ref[:, :] = mask


def _node(xf, cf, w_node, bn2, w1, w2):
    nb = (B * N) // BLKB
    return pl.pallas_call(
        _node_body,
        grid=(nb,),
        in_specs=[
            pl.BlockSpec((BLKB, 12), lambda i: (i, 0)),
            pl.BlockSpec((BLKB, 1), lambda i: (i, 0)),
            pl.BlockSpec((12, DN), lambda i: (0, 0)),
            pl.BlockSpec((1, DN), lambda i: (0, 0)),
            pl.BlockSpec((DN, DN), lambda i: (0, 0)),
            pl.BlockSpec((DN, DN), lambda i: (0, 0)),
        ],
        out_specs=[
            pl.BlockSpec((BLKB, DN), lambda i: (i, 0)),
            pl.BlockSpec((BLKB, DN), lambda i: (i, 0)),
            pl.BlockSpec((BLKB, DN // 2), lambda i: (i, 0)),
            pl.BlockSpec((BLKB, 1), lambda i: (i, 0)),
        ],
        out_shape=[
            jax.ShapeDtypeStruct((B * N, DN), jnp.float32),
            jax.ShapeDtypeStruct((B * N, DN), jnp.float32),
            jax.ShapeDtypeStruct((B * N, DN // 2), jnp.int32),
            jax.ShapeDtypeStruct((B * N, 1), jnp.float32),
        ],
    )(xf, cf, w_node, bn2, w1, w2)


def _edge_msg_body(vals_ref, idxp_ref, c_ref, mi_ref, we_ref, be_ref,
                   hj_ref, p1_ref, nh_ref, w3_ref, bm_ref, *rest):
    eh_ref, mij_ref, eidx_ref, out_ref = rest[-4:]
    sigma = 20.0 / 16.0
    centers = lax.broadcasted_iota(jnp.int32, (1, 16), 1).astype(
        jnp.float32) * (20.0 / 15.0)
    c_i = c_ref[0]
    m_i = mi_ref[0]
    vals = vals_ref[0]
    idxp = idxp_ref[0]
    cj = jnp.bitwise_and(idxp, 3)
    mij = m_i * (cj > 0).astype(jnp.float32)
    same = (c_i == cj).astype(jnp.float32)
    be = be_ref[0][None, :]
    w_same = we_ref[16:17, :]
    p1 = p1_ref[0]
    bm = bm_ref[0][None, :]
    acc = jnp.zeros((BLKC, DN), jnp.float32)
    G = 6
    for g in range(KN // G):
        ks = range(g * G, (g + 1) * G)
        dg = jnp.concatenate([vals[:, k:k + 1] for k in ks], axis=0)
        sg = jnp.concatenate([same[:, k:k + 1] for k in ks], axis=0)
        mg = jnp.concatenate([mij[:, k:k + 1] for k in ks], axis=0)
        q = dg - centers
        rbf = jnp.exp(-(q * q) / (2.0 * sigma * sigma))
        lin = jnp.dot(rbf, we_ref[0:16, :],
                      preferred_element_type=jnp.float32)
        lin = lin + sg * w_same + be
        ehg = _softplus(lin) * mg
        for j, k in enumerate(ks):
            eh_ref[0, :, k, :] = ehg[j * BLKC:(j + 1) * BLKC, :]
        hjg = jnp.concatenate([hj_ref[0, :, k, :] for k in ks], axis=0)
        hj_lo = lax.bitcast_convert_type(
            lax.shift_left(hjg, 16), jnp.float32)
        hj_hi = lax.bitcast_convert_type(
            jnp.bitwise_and(hjg, jnp.int32(-65536)), jnp.float32)
        z = jnp.dot(ehg.astype(jnp.bfloat16), w3_ref[:, :],
                    preferred_element_type=jnp.float32)
        z = z + jnp.concatenate([hj_lo, hj_hi], axis=1) + bm
        sp = _softplus(z + jnp.concatenate([p1] * G, axis=0)) * mg
        for j in range(G):
            acc = acc + sp[j * BLKC:(j + 1) * BLKC, :]
    mij_ref[0] = mij[:, :KN]
    eidx_ref[0] = lax.shift_right_logical(idxp, 2)[:, :KN]
    cnt = jnp.sum(mij[:, :KN], axis=1, keepdims=True)
    agg = acc / (cnt + 1e-6)
    out_ref[0] = (nh_ref[0] + agg) * m_i


def _edge_msg(bb, vals, idxp, cn, mi, w_edge, be2, hj, p1, nh, w3, bm2, *acc):
    in_specs = [
        pl.BlockSpec((1, BLKC, KP), lambda i: (0, i, 0)),
        pl.BlockSpec((1, BLKC, KP), lambda i: (0, i, 0)),
        pl.BlockSpec((1, BLKC, 1), lambda i, bb=bb: (bb, i, 0)),
        pl.BlockSpec((1, BLKC, 1), lambda i, bb=bb: (bb, i, 0)),
        pl.BlockSpec((17, DE), lambda i: (0, 0)),
        pl.BlockSpec((1, DE), lambda i: (0, 0)),
        pl.BlockSpec((1, BLKC, KN, DN // 2), lambda i: (0, i, 0, 0)),
        pl.BlockSpec((1, BLKC, DN), lambda i, bb=bb: (bb, i, 0)),
        pl.BlockSpec((1, BLKC, DN), lambda i, bb=bb: (bb, i, 0)),
        pl.BlockSpec((DE, DN), lambda i: (0, 0)),
        pl.BlockSpec((1, DN), lambda i: (0, 0)),
    ] + [pl.BlockSpec(memory_space=pl.ANY) for _ in acc]
    aliases = {11 + j: j for j in range(len(acc))}
    return pl.pallas_call(
        _edge_msg_body,
        grid=(N // BLKC,),
        in_specs=in_specs,
        out_specs=[
            pl.BlockSpec((1, BLKC, KN, DE), lambda i, bb=bb: (bb, i, 0, 0)),
            pl.BlockSpec((1, BLKC, KN), lambda i, bb=bb: (bb, i, 0)),
            pl.BlockSpec((1, BLKC, KN), lambda i, bb=bb: (bb, i, 0)),
            pl.BlockSpec((1, BLKC, DN), lambda i, bb=bb: (bb, i, 0)),
        ],
        out_shape=[
            jax.ShapeDtypeStruct((B, N, KN, DE), jnp.float32),
            jax.ShapeDtypeStruct((B, N, KN), jnp.float32),
            jax.ShapeDtypeStruct((B, N, KN), jnp.int32),
            jax.ShapeDtypeStruct((B, N, DN), jnp.float32),
        ],
        input_output_aliases=aliases,
    )(vals, idxp, cn, mi, w_edge, be2, hj, p1, nh, w3, bm2, *acc)


_NC, _NS = 2, 16
_NW = _NC * _NS
_EP = N * KN
_B_PER_W = _EP // _NW
_CH = 128
_NCH = _B_PER_W // _CH


_NBUF = 6


def _sc_gather_body(tbl_ref, idx_ref, out_ref, idx_v, *bufs_sems):
    bufs = bufs_sems[:_NBUF]
    gsem = bufs_sems[_NBUF:2 * _NBUF]
    osem = bufs_sems[2 * _NBUF:]
    wid = lax.axis_index("s") * _NC + lax.axis_index("c")
    base = wid * _B_PER_W
    pltpu.sync_copy(idx_ref.at[pl.ds(base, _B_PER_W)], idx_v)

    def fire_gather(ci):
        j = ci % _NBUF
        return pltpu.async_copy(
            tbl_ref.at[idx_v.at[pl.ds(ci * _CH, _CH)]], bufs[j], gsem[j])

    gpend, opend = {}, {}
    for ci in range(3):
        gpend[ci] = fire_gather(ci)
    for ci in range(_NCH):
        j = ci % _NBUF
        nxt = ci + 3
        if nxt < _NCH:
            if nxt - _NBUF >= 0:
                opend[nxt - _NBUF].wait()
            gpend[nxt] = fire_gather(nxt)
        gpend[ci].wait()
        opend[ci] = pltpu.async_copy(
            bufs[j], out_ref.at[pl.ds(base + ci * _CH, _CH)], osem[j])
    for ci in range(_NCH - _NBUF, _NCH):
        opend[ci].wait()


@functools.cache
def _sc_gather():
    return pl.kernel(
        _sc_gather_body,
        out_type=jax.ShapeDtypeStruct((_EP, DN // 2), jnp.int32),
        mesh=plsc.VectorSubcoreMesh(core_axis_name="c", subcore_axis_name="s",
                                    num_cores=_NC, num_subcores=_NS),
        scratch_types=(
            [pltpu.VMEM((_B_PER_W,), jnp.int32)]
            + [pltpu.VMEM((_CH, DN // 2), jnp.int32) for _ in range(_NBUF)]
            + [pltpu.SemaphoreType.DMA for _ in range(2 * _NBUF)]
        ),
    )


def _gather_rows(table, idxg):
    return _sc_gather()(table, idxg)


def kernel(X, C, W_node, b_node, W_edge, b_edge, W_msg, b_msg):
    X = X.astype(jnp.float32)
    C32 = C.astype(jnp.int32)

    centroid = X.mean(axis=2)
    cent = centroid.transpose(0, 2, 1)
    c2 = C32.reshape(B, 1, N)

    xf = X.reshape(B * N, 12)
    cf = C32.reshape(B * N, 1)
    w1 = W_msg[:DN, :]
    w2 = W_msg[DN:2 * DN, :]
    w3 = W_msg[2 * DN:, :].astype(jnp.bfloat16)
    nh, p1, p2, mi = _node(xf, cf, W_node, b_node.reshape(1, DN), w1, w2)

    cn = C32.reshape(B, N, 1)
    mi3 = mi.reshape(B, N, 1)
    p13 = p1.reshape(B, N, DN)
    nh3 = nh.reshape(B, N, DN)
    be2 = b_edge.reshape(1, DE)
    bm2 = b_msg.reshape(1, DN)

    knn_out = [_knn(bb, centroid, cent, c2) for bb in range(B)]
    hjs = [_gather_rows(p2, knn_out[bb][2].reshape(_EP)) for bb in range(B)]
    ems = ()
    for bb in range(B):
        ems = _edge_msg(bb, knn_out[bb][0], knn_out[bb][1], cn, mi3, W_edge,
                        be2, hjs[bb].reshape(1, N, KN, DN // 2), p13, nh3,
                        w3, bm2, *ems)
    eh, mij, eidx, node_h = ems

    return (node_h, eh, eidx, mi.reshape(B, N), mij)


# speedup vs baseline: 1.1077x; 1.0413x over previous
import functools

import jax
import jax.numpy as jnp
from jax import lax
from jax.experimental import pallas as pl
from jax.experimental.pallas import tpu as pltpu
from jax.experimental.pallas import tpu_sc as plsc

B = 4
N = 2048
KN = 30
KP = 32
DN = 256
DE = 128

BLKA = 512
BLKB = 512
BLKC = 256
BLKD = 128

def _softplus(x):
    return jnp.maximum(x, 0.0) + jnp.log1p(jnp.exp(-jnp.abs(x)))


def _knn_body(bb0, cen_ref, cent_ref, c_ref, vals_ref, idxp_ref, idxg_ref):
    b = bb0 + pl.program_id(0)
    i = pl.program_id(1)
    r0 = i * BLKA

    cols_c = c_ref[0]
    col_iota = lax.broadcasted_iota(jnp.int32, (1, N), 1)
    rows = r0 + lax.broadcasted_iota(jnp.int32, (BLKA, 1), 0)

    acc = None
    for d in range(3):
        diff = cen_ref[0, :, d:d + 1] - cent_ref[0, d:d + 1, :]
        acc = diff * diff if acc is None else acc + diff * diff
    dist = jnp.sqrt(acc + 1e-8)
    dist = dist + jnp.where(col_iota == rows, 1e6, 0.0)
    dist = dist + jnp.where(cols_c > 0, 0.0, 1e6)

    ip = col_iota * 4 + cols_c
    ip_b = jnp.broadcast_to(ip, (BLKA, N))
    lane = lax.broadcasted_iota(jnp.int32, (1, KP), 1)

    dc = dist
    vals = jnp.zeros((BLKA, KP), jnp.float32)
    idxp = jnp.zeros((BLKA, KP), jnp.int32)
    for k in range(KN):
        m = jnp.min(dc, axis=1, keepdims=True)
        pk = jnp.min(jnp.where(dc == m, ip_b, jnp.int32(1 << 30)),
                     axis=1, keepdims=True)
        dc = jnp.where(ip_b == pk, 1e9, dc)
        vals = jnp.where(lane == k, m, vals)
        idxp = jnp.where(lane == k, pk, idxp)
    vals_ref[0] = vals
    idxp_ref[0] = idxp
    idxg_ref[0] = (lax.shift_right_logical(idxp, 2) + b * N)[:, :KN]


_BH = 2


def _knn(bb0, cen, cent, c2):
    return pl.pallas_call(
        functools.partial(_knn_body, bb0),
        grid=(_BH, N // BLKA),
        in_specs=[
            pl.BlockSpec((1, BLKA, 3), lambda b, i, bb0=bb0: (bb0 + b, i, 0)),
            pl.BlockSpec((1, 3, N), lambda b, i, bb0=bb0: (bb0 + b, 0, 0)),
            pl.BlockSpec((1, 1, N), lambda b, i, bb0=bb0: (bb0 + b, 0, 0)),
        ],
        out_specs=[
            pl.BlockSpec((1, BLKA, KP), lambda b, i: (b, i, 0)),
            pl.BlockSpec((1, BLKA, KP), lambda b, i: (b, i, 0)),
            pl.BlockSpec((1, BLKA, KN), lambda b, i: (b, i, 0)),
        ],
        out_shape=[
            jax.ShapeDtypeStruct((_BH, N, KP), jnp.float32),
            jax.ShapeDtypeStruct((_BH, N, KP), jnp.int32),
            jax.ShapeDtypeStruct((_BH, N, KN), jnp.int32),
        ],
    )(cen, cent, c2)


def _node_body(x_ref, c_ref, wn_ref, bn_ref, w1_ref, w2_ref,
               nh_ref, p1_ref, p2_ref, m_ref):
    x = x_ref[:, :]
    mask = (c_ref[:, :] > 0).astype(jnp.float32)
    h = jnp.dot(x, wn_ref[:, :], preferred_element_type=jnp.float32)
    h = _softplus(h + bn_ref[0][None, :]) * mask
    nh_ref[:, :] = h
    p1_ref[:, :] = jnp.dot(h, w1_ref[:, :], preferred_element_type=jnp.float32)
    p2 = jnp.dot(h, w2_ref[:, :], preferred_element_type=jnp.float32)
    def _rne_hi16(x):
        bits = lax.bitcast_convert_type(x, jnp.int32)
        lsb = jnp.bitwise_and(lax.shift_right_logical(bits, 16), 1)
        return jnp.bitwise_and(bits + 32767 + lsb, jnp.int32(-65536))

    lo = lax.shift_right_logical(_rne_hi16(p2[:, :DN // 2]), 16)
    hi = _rne_hi16(p2[:, DN // 2:])
    p2_ref[:, :] = jnp.bitwise_or(hi, lo)
    m_ref[:, :] = mask


def _node(xf, cf, w_node, bn2, w1, w2):
    nb = (B * N) // BLKB
    return pl.pallas_call(
        _node_body,
        grid=(nb,),
        in_specs=[
            pl.BlockSpec((BLKB, 12), lambda i: (i, 0)),
            pl.BlockSpec((BLKB, 1), lambda i: (i, 0)),
            pl.BlockSpec((12, DN), lambda i: (0, 0)),
            pl.BlockSpec((1, DN), lambda i: (0, 0)),
            pl.BlockSpec((DN, DN), lambda i: (0, 0)),
            pl.BlockSpec((DN, DN), lambda i: (0, 0)),
        ],
        out_specs=[
            pl.BlockSpec((BLKB, DN), lambda i: (i, 0)),
            pl.BlockSpec((BLKB, DN), lambda i: (i, 0)),
            pl.BlockSpec((BLKB, DN // 2), lambda i: (i, 0)),
            pl.BlockSpec((BLKB, 1), lambda i: (i, 0)),
        ],
        out_shape=[
            jax.ShapeDtypeStruct((B * N, DN), jnp.float32),
            jax.ShapeDtypeStruct((B * N, DN), jnp.float32),
            jax.ShapeDtypeStruct((B * N, DN // 2), jnp.int32),
            jax.ShapeDtypeStruct((B * N, 1), jnp.float32),
        ],
    )(xf, cf, w_node, bn2, w1, w2)


def _edge_msg_body(vals_ref, idxp_ref, c_ref, mi_ref, we_ref, be_ref,
                   hj_ref, p1_ref, nh_ref, w3_ref, bm_ref, *rest):
    eh_ref, mij_ref, eidx_ref, out_ref = rest[-4:]
    sigma = 20.0 / 16.0
    centers = lax.broadcasted_iota(jnp.int32, (1, 16), 1).astype(
        jnp.float32) * (20.0 / 15.0)
    c_i = c_ref[0]
    m_i = mi_ref[0]
    vals = vals_ref[0]
    idxp = idxp_ref[0]
    cj = jnp.bitwise_and(idxp, 3)
    mij = m_i * (cj > 0).astype(jnp.float32)
    same = (c_i == cj).astype(jnp.float32)
    be = be_ref[0][None, :]
    w_same = we_ref[16:17, :]
    p1 = p1_ref[0]
    bm = bm_ref[0][None, :]
    acc = jnp.zeros((BLKC, DN), jnp.float32)
    G = 6
    for g in range(KN // G):
        ks = range(g * G, (g + 1) * G)
        dg = jnp.concatenate([vals[:, k:k + 1] for k in ks], axis=0)
        sg = jnp.concatenate([same[:, k:k + 1] for k in ks], axis=0)
        mg = jnp.concatenate([mij[:, k:k + 1] for k in ks], axis=0)
        q = dg - centers
        rbf = jnp.exp(-(q * q) / (2.0 * sigma * sigma))
        lin = jnp.dot(rbf, we_ref[0:16, :],
                      preferred_element_type=jnp.float32)
        lin = lin + sg * w_same + be
        ehg = _softplus(lin) * mg
        for j, k in enumerate(ks):
            eh_ref[0, :, k, :] = ehg[j * BLKC:(j + 1) * BLKC, :]
        hjg = jnp.concatenate([hj_ref[0, :, k, :] for k in ks], axis=0)
        hj_lo = lax.bitcast_convert_type(
            lax.shift_left(hjg, 16), jnp.float32)
        hj_hi = lax.bitcast_convert_type(
            jnp.bitwise_and(hjg, jnp.int32(-65536)), jnp.float32)
        z = jnp.dot(ehg.astype(jnp.bfloat16), w3_ref[:, :],
                    preferred_element_type=jnp.float32)
        z = z + jnp.concatenate([hj_lo, hj_hi], axis=1) + bm
        sp = _softplus(z + jnp.concatenate([p1] * G, axis=0)) * mg
        for j in range(G):
            acc = acc + sp[j * BLKC:(j + 1) * BLKC, :]
    mij_ref[0] = mij[:, :KN]
    eidx_ref[0] = lax.shift_right_logical(idxp, 2)[:, :KN]
    cnt = jnp.sum(mij[:, :KN], axis=1, keepdims=True)
    agg = acc / (cnt + 1e-6)
    out_ref[0] = (nh_ref[0] + agg) * m_i


def _edge_msg(bb, vals, idxp, cn, mi, w_edge, be2, hj, p1, nh, w3, bm2, *acc):
    in_specs = [
        pl.BlockSpec((1, BLKC, KP), lambda b, i: (b, i, 0)),
        pl.BlockSpec((1, BLKC, KP), lambda b, i: (b, i, 0)),
        pl.BlockSpec((1, BLKC, 1), lambda b, i, bb=bb: (bb + b, i, 0)),
        pl.BlockSpec((1, BLKC, 1), lambda b, i, bb=bb: (bb + b, i, 0)),
        pl.BlockSpec((17, DE), lambda b, i: (0, 0)),
        pl.BlockSpec((1, DE), lambda b, i: (0, 0)),
        pl.BlockSpec((1, BLKC, KN, DN // 2), lambda b, i: (b, i, 0, 0)),
        pl.BlockSpec((1, BLKC, DN), lambda b, i, bb=bb: (bb + b, i, 0)),
        pl.BlockSpec((1, BLKC, DN), lambda b, i, bb=bb: (bb + b, i, 0)),
        pl.BlockSpec((DE, DN), lambda b, i: (0, 0)),
        pl.BlockSpec((1, DN), lambda b, i: (0, 0)),
    ] + [pl.BlockSpec(memory_space=pl.ANY) for _ in acc]
    aliases = {11 + j: j for j in range(len(acc))}
    return pl.pallas_call(
        _edge_msg_body,
        grid=(_BH, N // BLKC),
        in_specs=in_specs,
        out_specs=[
            pl.BlockSpec((1, BLKC, KN, DE),
                         lambda b, i, bb=bb: (bb + b, i, 0, 0)),
            pl.BlockSpec((1, BLKC, KN), lambda b, i, bb=bb: (bb + b, i, 0)),
            pl.BlockSpec((1, BLKC, KN), lambda b, i, bb=bb: (bb + b, i, 0)),
            pl.BlockSpec((1, BLKC, DN), lambda b, i, bb=bb: (bb + b, i, 0)),
        ],
        out_shape=[
            jax.ShapeDtypeStruct((B, N, KN, DE), jnp.float32),
            jax.ShapeDtypeStruct((B, N, KN), jnp.float32),
            jax.ShapeDtypeStruct((B, N, KN), jnp.int32),
            jax.ShapeDtypeStruct((B, N, DN), jnp.float32),
        ],
        input_output_aliases=aliases,
    )(vals, idxp, cn, mi, w_edge, be2, hj, p1, nh, w3, bm2, *acc)


_NC, _NS = 2, 16
_NW = _NC * _NS
_EP = _BH * N * KN
_B_PER_W = _EP // _NW
_CH = 128
_NCH = _B_PER_W // _CH


_NBUF = 6


def _sc_gather_body(tbl_ref, idx_ref, out_ref, idx_v, *bufs_sems):
    bufs = bufs_sems[:_NBUF]
    gsem = bufs_sems[_NBUF:2 * _NBUF]
    osem = bufs_sems[2 * _NBUF:]
    wid = lax.axis_index("s") * _NC + lax.axis_index("c")
    base = wid * _B_PER_W
    pltpu.sync_copy(idx_ref.at[pl.ds(base, _B_PER_W)], idx_v)

    def fire_gather(ci):
        j = ci % _NBUF
        return pltpu.async_copy(
            tbl_ref.at[idx_v.at[pl.ds(ci * _CH, _CH)]], bufs[j], gsem[j])

    gpend, opend = {}, {}
    for ci in range(3):
        gpend[ci] = fire_gather(ci)
    for ci in range(_NCH):
        j = ci % _NBUF
        nxt = ci + 3
        if nxt < _NCH:
            if nxt - _NBUF >= 0:
                opend[nxt - _NBUF].wait()
            gpend[nxt] = fire_gather(nxt)
        gpend[ci].wait()
        opend[ci] = pltpu.async_copy(
            bufs[j], out_ref.at[pl.ds(base + ci * _CH, _CH)], osem[j])
    for ci in range(_NCH - _NBUF, _NCH):
        opend[ci].wait()


@functools.cache
def _sc_gather():
    return pl.kernel(
        _sc_gather_body,
        out_type=jax.ShapeDtypeStruct((_EP, DN // 2), jnp.int32),
        mesh=plsc.VectorSubcoreMesh(core_axis_name="c", subcore_axis_name="s",
                                    num_cores=_NC, num_subcores=_NS),
        scratch_types=(
            [pltpu.VMEM((_B_PER_W,), jnp.int32)]
            + [pltpu.VMEM((_CH, DN // 2), jnp.int32) for _ in range(_NBUF)]
            + [pltpu.SemaphoreType.DMA for _ in range(2 * _NBUF)]
        ),
    )


def _gather_rows(table, idxg):
    return _sc_gather()(table, idxg)


def kernel(X, C, W_node, b_node, W_edge, b_edge, W_msg, b_msg):
    X = X.astype(jnp.float32)
    C32 = C.astype(jnp.int32)

    centroid = X.mean(axis=2)
    cent = centroid.transpose(0, 2, 1)
    c2 = C32.reshape(B, 1, N)

    xf = X.reshape(B * N, 12)
    cf = C32.reshape(B * N, 1)
    w1 = W_msg[:DN, :]
    w2 = W_msg[DN:2 * DN, :]
    w3 = W_msg[2 * DN:, :].astype(jnp.bfloat16)
    nh, p1, p2, mi = _node(xf, cf, W_node, b_node.reshape(1, DN), w1, w2)

    cn = C32.reshape(B, N, 1)
    mi3 = mi.reshape(B, N, 1)
    p13 = p1.reshape(B, N, DN)
    nh3 = nh.reshape(B, N, DN)
    be2 = b_edge.reshape(1, DE)
    bm2 = b_msg.reshape(1, DN)

    knn_out = [_knn(bb, centroid, cent, c2) for bb in range(0, B, _BH)]
    hjs = [_gather_rows(p2, ko[2].reshape(_EP)) for ko in knn_out]
    ems = ()
    for h, bb in enumerate(range(0, B, _BH)):
        ems = _edge_msg(bb, knn_out[h][0], knn_out[h][1], cn, mi3, W_edge,
                        be2, hjs[h].reshape(_BH, N, KN, DN // 2), p13, nh3,
                        w3, bm2, *ems)
    eh, mij, eidx, node_h = ems

    return (node_h, eh, eidx, mi.reshape(B, N), mij)
